# baseline (speedup 1.0000x reference)
import functools

import jax
import jax.numpy as jnp
import numpy as np
from jax import lax
from jax.experimental import pallas as pl
from jax.experimental.pallas import tpu as pltpu
from jax.experimental.pallas import tpu_sc as plsc

NL = 5
N = 10000
NP = 10240
E = 160000
D = 300
PD = 320
HALF = PD // 2
DH = 640
G = 256
ECS = 32
CPP = 25
MAXC = 325
NPHS = MAXC // CPP
EPS = 16 * MAXC * ECS
NCHS = EPS // ECS
ECC = 128
MAXCC = 40
EPC = 32 * MAXCC * ECC
NCHC = EPC // ECC
NS = 16
NPS = 10016
SENT = 10000
RPT = NPS // NS
XPAD = NP - NPS
BLK = 1280
NBLK = NP // BLK
ONES_COL = 304


def _sc_counts_body(bond_hbm, bdir_hbm, dst_hbm, tab_hbm, zero_hbm, out_hbm,
                    bd_v, dd_v, di_v, r0, r1, r2, r3, acc_s, gsem, ssem):
    c = lax.axis_index("c")
    s = lax.axis_index("s")
    w = s * 2 + c
    pltpu.sync_copy(zero_hbm.at[pl.ds(s * RPT, RPT)],
                    acc_s.at[pl.ds(s * RPT, RPT)])
    e0 = w * (MAXCC * ECC)
    pltpu.sync_copy(bond_hbm.at[pl.ds(e0, MAXCC * ECC)], bd_v)
    pltpu.sync_copy(bdir_hbm.at[pl.ds(e0, MAXCC * ECC)], dd_v)
    pltpu.sync_copy(dst_hbm.at[pl.ds(w * MAXCC, MAXCC)], di_v)

    w24 = jnp.full((16,), w * 24, dtype=jnp.int32)

    def combo(j, carry):
        sl = pl.ds(j * 16, 16)
        bd_v[sl] = bd_v[sl] * 3 + dd_v[sl] + w24
        return carry

    lax.fori_loop(0, MAXCC * ECC // 16, combo, 0)
    plsc.subcore_barrier()
    rows = (r0, r1, r2, r3)
    pltpu.async_copy(tab_hbm.at[bd_v.at[pl.ds(0, ECC)]], r0, gsem)
    pltpu.async_copy(tab_hbm.at[bd_v.at[pl.ds(ECC, ECC)]], r1, gsem)

    def step(i, carry):
        for u in range(4):
            k = 4 * i + u
            pltpu.make_async_copy(zero_hbm.at[pl.ds(0, ECC)], rows[u],
                                  gsem).wait()

            @pl.when(k >= 2)
            def _():
                pltpu.make_async_copy(zero_hbm.at[pl.ds(0, ECC)], rows[u],
                                      ssem).wait()

            pltpu.async_copy(rows[u], acc_s.at[di_v.at[k]], ssem, add=True)

            @pl.when(k + 2 < MAXCC)
            def _():
                pltpu.async_copy(
                    tab_hbm.at[bd_v.at[pl.ds((k + 2) * ECC, ECC)]],
                    rows[(u + 2) % 4], gsem)
        return carry

    lax.fori_loop(0, MAXCC // 4, step, 0)
    pltpu.make_async_copy(zero_hbm.at[pl.ds(0, ECC)], r0, ssem).wait()
    pltpu.make_async_copy(zero_hbm.at[pl.ds(0, ECC)], r0, ssem).wait()
    plsc.subcore_barrier()
    pltpu.sync_copy(acc_s.at[pl.ds(s * RPT, RPT)],
                    out_hbm.at[c, pl.ds(s * RPT, RPT)])

    @pl.when(s == NS - 1)
    def _():
        pltpu.sync_copy(acc_s.at[pl.ds(0, XPAD)],
                        out_hbm.at[c, pl.ds(NPS, XPAD)])


_sc_counts = functools.partial(
    pl.kernel,
    out_type=jax.ShapeDtypeStruct((2, NP, 16), jnp.float32),
    compiler_params=pltpu.CompilerParams(use_tc_tiling_on_sc=False),
    mesh=plsc.VectorSubcoreMesh(core_axis_name="c", subcore_axis_name="s"),
    scratch_types=[
        pltpu.VMEM((MAXCC * ECC,), jnp.int32),
        pltpu.VMEM((MAXCC * ECC,), jnp.int32),
        pltpu.VMEM((MAXCC, ECC), jnp.int32),
        pltpu.VMEM((ECC, 16), jnp.float32),
        pltpu.VMEM((ECC, 16), jnp.float32),
        pltpu.VMEM((ECC, 16), jnp.float32),
        pltpu.VMEM((ECC, 16), jnp.float32),
        pltpu.VMEM_SHARED((NPS, 16), jnp.float32),
        pltpu.SemaphoreType.DMA,
        pltpu.SemaphoreType.DMA,
    ],
)(_sc_counts_body)


def _sc_spmm_body(src01_hbm, dst_hbm, hflat_hbm, init_hbm, out_hbm,
                  si0, si1, di0, di1, r0, r1, r2, r3, r4, acc_s,
                  gsem, ssem, isem):
    c = lax.axis_index("c")
    s = lax.axis_index("s")
    pltpu.sync_copy(init_hbm.at[c, pl.ds(s * RPT, RPT)],
                    acc_s.at[pl.ds(s * RPT, RPT)])
    e0 = s * (MAXC * ECS)
    c0 = s * MAXC
    PE = CPP * ECS
    pltpu.sync_copy(src01_hbm.at[c, pl.ds(e0, PE)], si0)
    pltpu.sync_copy(dst_hbm.at[pl.ds(c0, CPP)], di0)
    plsc.subcore_barrier()
    rows = (r0, r1, r2, r3, r4)
    sis = (si0, si1)
    dis = (di0, di1)
    pltpu.async_copy(hflat_hbm.at[si0.at[pl.ds(0, ECS)]], r0, gsem)
    pltpu.async_copy(hflat_hbm.at[si0.at[pl.ds(ECS, ECS)]], r1, gsem)
    pltpu.async_copy(hflat_hbm.at[si0.at[pl.ds(2 * ECS, ECS)]], r2, gsem)

    def step(i, carry):
        for u in range(5):
            k = 5 * i + u
            p = k // CPP
            kr = k - p * CPP
            pltpu.make_async_copy(hflat_hbm.at[pl.ds(0, ECS)], rows[u],
                                  gsem).wait()

            @pl.when(k >= 2)
            def _():
                pltpu.make_async_copy(hflat_hbm.at[pl.ds(0, ECS)], rows[u],
                                      ssem).wait()

            for q in range(2):
                @pl.when(p % 2 == q)
                def _():
                    pltpu.async_copy(rows[u], acc_s.at[dis[q].at[kr]],
                                     ssem, add=True)

            @pl.when((kr == 2) & (p + 1 < NPHS))
            def _():
                for q in range(2):
                    @pl.when((p + 1) % 2 == q)
                    def _():
                        pltpu.async_copy(
                            src01_hbm.at[c, pl.ds(e0 + (p + 1) * PE, PE)],
                            sis[q], isem)
                        pltpu.async_copy(
                            dst_hbm.at[pl.ds(c0 + (p + 1) * CPP, CPP)],
                            dis[q], isem)

            @pl.when((kr == CPP - 3) & (p + 1 < NPHS))
            def _():
                pltpu.make_async_copy(src01_hbm.at[0, pl.ds(0, PE)], si0,
                                      isem).wait()
                pltpu.make_async_copy(dst_hbm.at[pl.ds(0, CPP)], di0,
                                      isem).wait()

            @pl.when(k + 3 < MAXC)
            def _():
                k2 = k + 3
                p2 = k2 // CPP
                o2 = (k2 - p2 * CPP) * ECS
                for q in range(2):
                    @pl.when(p2 % 2 == q)
                    def _():
                        pltpu.async_copy(
                            hflat_hbm.at[sis[q].at[pl.ds(o2, ECS)]],
                            rows[(u + 3) % 5], gsem)
        return carry

    lax.fori_loop(0, MAXC // 5, step, 0)
    pltpu.make_async_copy(hflat_hbm.at[pl.ds(0, ECS)], r0, ssem).wait()
    pltpu.make_async_copy(hflat_hbm.at[pl.ds(0, ECS)], r0, ssem).wait()
    plsc.subcore_barrier()
    pltpu.sync_copy(acc_s.at[pl.ds(s * RPT, RPT)],
                    out_hbm.at[c, pl.ds(s * RPT, RPT)])

    @pl.when(s == NS - 1)
    def _():
        pltpu.sync_copy(acc_s.at[pl.ds(0, XPAD)],
                        out_hbm.at[c, pl.ds(NPS, XPAD)])


_sc_spmm = functools.partial(
    pl.kernel,
    out_type=jax.ShapeDtypeStruct((2, NP, HALF), jnp.float32),
    compiler_params=pltpu.CompilerParams(use_tc_tiling_on_sc=False),
    mesh=plsc.VectorSubcoreMesh(core_axis_name="c", subcore_axis_name="s"),
    scratch_types=[
        pltpu.VMEM((CPP * ECS,), jnp.int32),
        pltpu.VMEM((CPP * ECS,), jnp.int32),
        pltpu.VMEM((CPP, ECS), jnp.int32),
        pltpu.VMEM((CPP, ECS), jnp.int32),
        pltpu.VMEM((ECS, HALF), jnp.float32),
        pltpu.VMEM((ECS, HALF), jnp.float32),
        pltpu.VMEM((ECS, HALF), jnp.float32),
        pltpu.VMEM((ECS, HALF), jnp.float32),
        pltpu.VMEM((ECS, HALF), jnp.float32),
        pltpu.VMEM_SHARED((NPS, HALF), jnp.float32),
        pltpu.SemaphoreType.DMA,
        pltpu.SemaphoreType.DMA,
        pltpu.SemaphoreType.DMA,
    ],
)(_sc_spmm_body)


def _embed_body(atom_ref, chir_ref, embA_ref, embC_ref, h_ref):
    a = atom_ref[0, 0, :]
    ch = chir_ref[0, 0, :]
    oa = (a[:, None] == lax.broadcasted_iota(jnp.int32, (BLK, 128), 1)
          ).astype(jnp.float32)
    oc = (ch[:, None] == lax.broadcasted_iota(jnp.int32, (BLK, 8), 1)
          ).astype(jnp.float32)
    h = oa @ embA_ref[...] + oc @ embC_ref[...]
    h_ref[0] = h[:, :HALF]
    h_ref[1] = h[:, HALF:]


def _mlp_body(agg_ref, c9_ref, w1_ref, b1_ref, w2_ref, b2_ref, sc_ref,
              bb_ref, e9_ref, sl_ref, h_ref):
    cnt = c9_ref[0] + c9_ref[1]
    x = (jnp.concatenate([agg_ref[0], agg_ref[1]], axis=1)
         + cnt @ e9_ref[...] + sl_ref[...])
    hmid = jnp.maximum(x @ w1_ref[...] + b1_ref[...], 0.0)
    h = (hmid @ w2_ref[...] + b2_ref[...]) * sc_ref[...] + bb_ref[...]
    h = jnp.maximum(h, 0.0)
    h_ref[0] = h[:, :HALF]
    h_ref[1] = h[:, HALF:]


def _final_body(agg_ref, c9_ref, w1_ref, b1_ref, w2_ref, b2_ref, sc_ref,
                bb_ref, e9_ref, sl_ref, gid_ref, out_ref, sums_ref):
    i = pl.program_id(0)
    cnt = c9_ref[0] + c9_ref[1]
    x = (jnp.concatenate([agg_ref[0], agg_ref[1]], axis=1)
         + cnt @ e9_ref[...] + sl_ref[...])
    hmid = jnp.maximum(x @ w1_ref[...] + b1_ref[...], 0.0)
    h = (hmid @ w2_ref[...] + b2_ref[...]) * sc_ref[...] + bb_ref[...]
    lane = lax.broadcasted_iota(jnp.int32, (BLK, PD), 1)
    h_aug = jnp.where(lane == ONES_COL, 1.0, h)
    gids = gid_ref[0, 0, :]
    oh = (gids[:, None] == lax.broadcasted_iota(jnp.int32, (BLK, G), 1)
          ).astype(jnp.float32)
    part = lax.dot_general(oh, h_aug, (((0,), (0,)), ((), ())),
                           preferred_element_type=jnp.float32)

    @pl.when(i == 0)
    def _():
        sums_ref[...] = part

    @pl.when(i > 0)
    def _():
        sums_ref[...] = sums_ref[...] + part

    @pl.when(i == NBLK - 1)
    def _():
        sums = sums_ref[...]
        cntg = jnp.maximum(sums[:, ONES_COL:ONES_COL + 1], 1.0)
        avg = sums[:, 0:D] / cntg
        out_ref[...] = jnp.concatenate([avg, sums[:, 0:D]], axis=1)


def _full(shape):
    return pl.BlockSpec(shape, lambda i: tuple(0 for _ in shape))


_row3 = pl.BlockSpec((1, 1, BLK), lambda i: (i, 0, 0))
_hspec = pl.BlockSpec((2, BLK, HALF), lambda i: (0, i, 0))
_c9spec = pl.BlockSpec((2, BLK, 16), lambda i: (0, i, 0))

_tc_embed = pl.pallas_call(
    _embed_body,
    grid=(NBLK,),
    in_specs=[_row3, _row3, _full((128, PD)), _full((8, PD))],
    out_specs=_hspec,
    out_shape=jax.ShapeDtypeStruct((2, NP, HALF), jnp.float32),
)

_tc_mlp = pl.pallas_call(
    _mlp_body,
    grid=(NBLK,),
    in_specs=[_hspec, _c9spec, _full((PD, DH)), _full((1, DH)),
              _full((DH, PD)), _full((1, PD)), _full((1, PD)),
              _full((1, PD)), _full((16, PD)), _full((1, PD))],
    out_specs=_hspec,
    out_shape=jax.ShapeDtypeStruct((2, NP, HALF), jnp.float32),
)

_tc_final = pl.pallas_call(
    _final_body,
    grid=(NBLK,),
    in_specs=[_hspec, _c9spec, _full((PD, DH)), _full((1, DH)),
              _full((DH, PD)), _full((1, PD)), _full((1, PD)),
              _full((1, PD)), _full((16, PD)), _full((1, PD)), _row3],
    out_specs=pl.BlockSpec((G, 2 * D), lambda i: (0, 0)),
    out_shape=jax.ShapeDtypeStruct((G, 2 * D), jnp.float32),
    scratch_shapes=[pltpu.VMEM((G, PD), jnp.float32)],
)

_TAB = np.zeros((24, 16), np.float32)
for _b in range(6):
    for _d in range(3):
        _TAB[_b * 3 + _d, _b] = 1.0
        _TAB[_b * 3 + _d, 6 + _d] = 1.0


def kernel(atomic_number, chirality_type, edge_index, bond_type,
           bond_direction_type, graph_ids, node_emb_atomic,
           node_emb_chirality, edge_emb_bond, edge_emb_dir,
           W1, b1, W2, b2, bn_scale, bn_bias):
    f32, i32 = jnp.float32, jnp.int32
    pad = NP - N
    srcp = jnp.pad(edge_index[0].astype(i32), (0, EPS - E))
    src01 = jnp.stack([srcp, srcp + NP])
    dst = jnp.pad(edge_index[1].astype(i32), (0, EPS - E),
                  constant_values=SENT).reshape(NCHS, ECS)
    dstc = jnp.pad(edge_index[1].astype(i32), (0, EPC - E),
                   constant_values=SENT).reshape(NCHC, ECC)
    bond = jnp.pad(bond_type.astype(i32), (0, EPC - E))
    bdir = jnp.pad(bond_direction_type.astype(i32), (0, EPC - E))
    atom3 = jnp.pad(atomic_number.astype(i32), (0, pad)).reshape(NBLK, 1, BLK)
    chir3 = jnp.pad(chirality_type.astype(i32), (0, pad)).reshape(NBLK, 1, BLK)
    gid3 = jnp.pad(graph_ids.astype(i32), (0, pad),
                   constant_values=G + 7).reshape(NBLK, 1, BLK)
    embA = jnp.pad(node_emb_atomic.astype(f32), ((0, 8), (0, PD - D)))
    embC = jnp.pad(node_emb_chirality.astype(f32), ((0, 5), (0, PD - D)))
    W1p = jnp.pad(W1.astype(f32), ((0, 0), (0, PD - D), (0, DH - 2 * D)))
    b1p = jnp.pad(b1.astype(f32), ((0, 0), (0, DH - 2 * D))).reshape(NL, 1, DH)
    W2p = jnp.pad(W2.astype(f32), ((0, 0), (0, DH - 2 * D), (0, PD - D)))
    b2p = jnp.pad(b2.astype(f32), ((0, 0), (0, PD - D))).reshape(NL, 1, PD)
    scp = jnp.pad(bn_scale.astype(f32), ((0, 0), (0, PD - D))).reshape(NL, 1, PD)
    bbp = jnp.pad(bn_bias.astype(f32), ((0, 0), (0, PD - D))).reshape(NL, 1, PD)
    e9 = jnp.pad(
        jnp.concatenate([edge_emb_bond.astype(f32), edge_emb_dir.astype(f32)],
                        axis=1), ((0, 0), (0, 7), (0, PD - D)))
    slf = jnp.pad((edge_emb_bond[:, 4] + edge_emb_dir[:, 0]).astype(f32),
                  ((0, 0), (0, PD - D))).reshape(NL, 1, PD)
    tab = jnp.tile(jnp.asarray(_TAB), (32, 1))
    zeros16 = jnp.zeros((NP, 16), f32)

    counts = _sc_counts(bond, bdir, dstc, tab, zeros16)
    h2 = _tc_embed(atom3, chir3, embA, embC)
    for l in range(NL):
        agg = _sc_spmm(src01, dst, h2.reshape(2 * NP, HALF), h2)
        if l < NL - 1:
            h2 = _tc_mlp(agg, counts, W1p[l], b1p[l], W2p[l], b2p[l],
                         scp[l], bbp[l], e9[l], slf[l])
        else:
            res = _tc_final(agg, counts, W1p[l], b1p[l], W2p[l], b2p[l],
                            scp[l], bbp[l], e9[l], slf[l], gid3)
    return res


# speedup vs baseline: 2.0442x; 2.0442x over previous
import functools

import jax
import jax.numpy as jnp
import numpy as np
from jax import lax
from jax.experimental import pallas as pl
from jax.experimental.pallas import tpu as pltpu
from jax.experimental.pallas import tpu_sc as plsc

NL = 5
N = 10000
NP = 10240
E = 160000
D = 300
PD = 320
HALF = PD // 2
DH = 640
G = 256
ECS = 40
CPP = 28
MAXC = 252
NPHS = MAXC // CPP
EPS = 16 * MAXC * ECS
NCHS = EPS // ECS
ECC = 128
MAXCC = 40
EPC = 32 * MAXCC * ECC
NCHC = EPC // ECC
NS = 16
NPS = 10016
SENT = 10000
RPT = NPS // NS
XPAD = NP - NPS
BLK = 1280
NBLK = NP // BLK
ONES_COL = 304


def _sc_counts_body(bond_hbm, bdir_hbm, dst_hbm, tab_hbm, zero_hbm, out_hbm,
                    bd_v, dd_v, di_v, r0, r1, r2, r3, acc_s, gsem, ssem):
    c = lax.axis_index("c")
    s = lax.axis_index("s")
    w = s * 2 + c
    pltpu.sync_copy(zero_hbm.at[pl.ds(s * RPT, RPT)],
                    acc_s.at[pl.ds(s * RPT, RPT)])
    e0 = w * (MAXCC * ECC)
    pltpu.sync_copy(bond_hbm.at[pl.ds(e0, MAXCC * ECC)], bd_v)
    pltpu.sync_copy(bdir_hbm.at[pl.ds(e0, MAXCC * ECC)], dd_v)
    pltpu.sync_copy(dst_hbm.at[pl.ds(w * MAXCC, MAXCC)], di_v)

    w24 = jnp.full((16,), w * 24, dtype=jnp.int32)

    def combo(j, carry):
        sl = pl.ds(j * 16, 16)
        bd_v[sl] = bd_v[sl] * 3 + dd_v[sl] + w24
        return carry

    lax.fori_loop(0, MAXCC * ECC // 16, combo, 0)
    plsc.subcore_barrier()
    rows = (r0, r1, r2, r3)
    pltpu.async_copy(tab_hbm.at[bd_v.at[pl.ds(0, ECC)]], r0, gsem)
    pltpu.async_copy(tab_hbm.at[bd_v.at[pl.ds(ECC, ECC)]], r1, gsem)

    def step(i, carry):
        for u in range(4):
            k = 4 * i + u
            pltpu.make_async_copy(zero_hbm.at[pl.ds(0, ECC)], rows[u],
                                  gsem).wait()

            @pl.when(k >= 2)
            def _():
                pltpu.make_async_copy(zero_hbm.at[pl.ds(0, ECC)], rows[u],
                                      ssem).wait()

            pltpu.async_copy(rows[u], acc_s.at[di_v.at[k]], ssem, add=True)

            @pl.when(k + 2 < MAXCC)
            def _():
                pltpu.async_copy(
                    tab_hbm.at[bd_v.at[pl.ds((k + 2) * ECC, ECC)]],
                    rows[(u + 2) % 4], gsem)
        return carry

    lax.fori_loop(0, MAXCC // 4, step, 0)
    pltpu.make_async_copy(zero_hbm.at[pl.ds(0, ECC)], r0, ssem).wait()
    pltpu.make_async_copy(zero_hbm.at[pl.ds(0, ECC)], r0, ssem).wait()
    plsc.subcore_barrier()
    pltpu.sync_copy(acc_s.at[pl.ds(s * RPT, RPT)],
                    out_hbm.at[c, pl.ds(s * RPT, RPT)])

    @pl.when(s == NS - 1)
    def _():
        pltpu.sync_copy(acc_s.at[pl.ds(0, XPAD)],
                        out_hbm.at[c, pl.ds(NPS, XPAD)])


_sc_counts = functools.partial(
    pl.kernel,
    out_type=jax.ShapeDtypeStruct((2, NP, 16), jnp.float32),
    compiler_params=pltpu.CompilerParams(use_tc_tiling_on_sc=False),
    mesh=plsc.VectorSubcoreMesh(core_axis_name="c", subcore_axis_name="s"),
    scratch_types=[
        pltpu.VMEM((MAXCC * ECC,), jnp.int32),
        pltpu.VMEM((MAXCC * ECC,), jnp.int32),
        pltpu.VMEM((MAXCC, ECC), jnp.int32),
        pltpu.VMEM((ECC, 16), jnp.float32),
        pltpu.VMEM((ECC, 16), jnp.float32),
        pltpu.VMEM((ECC, 16), jnp.float32),
        pltpu.VMEM((ECC, 16), jnp.float32),
        pltpu.VMEM_SHARED((NPS, 16), jnp.float32),
        pltpu.SemaphoreType.DMA,
        pltpu.SemaphoreType.DMA,
    ],
)(_sc_counts_body)


def _sc_spmm_body(src01_hbm, dst_hbm, hflat_hbm, init_hbm, out_hbm,
                  si0, si1, di0, di1, r0, r1, r2, r3, acc_s,
                  gsem, ssem, isem):
    c = lax.axis_index("c")
    s = lax.axis_index("s")
    pltpu.sync_copy(init_hbm.at[c, pl.ds(s * RPT, RPT)],
                    acc_s.at[pl.ds(s * RPT, RPT)])
    e0 = s * (MAXC * ECS)
    c0 = s * MAXC
    PE = CPP * ECS
    pltpu.sync_copy(src01_hbm.at[c, pl.ds(e0, PE)], si0)
    pltpu.sync_copy(dst_hbm.at[pl.ds(c0, CPP)], di0)
    plsc.subcore_barrier()
    rows = (r0, r1, r2, r3)
    sis = (si0, si1)
    dis = (di0, di1)
    pltpu.async_copy(hflat_hbm.at[si0.at[pl.ds(0, ECS)]], r0, gsem)
    pltpu.async_copy(hflat_hbm.at[si0.at[pl.ds(ECS, ECS)]], r1, gsem)

    def step(i, carry):
        for u in range(4):
            k = 4 * i + u
            p = k // CPP
            kr = k - p * CPP
            pltpu.make_async_copy(hflat_hbm.at[pl.ds(0, ECS)], rows[u],
                                  gsem).wait()

            @pl.when(k >= 2)
            def _():
                pltpu.make_async_copy(hflat_hbm.at[pl.ds(0, ECS)], rows[u],
                                      ssem).wait()

            for q in range(2):
                @pl.when(p % 2 == q)
                def _():
                    pltpu.async_copy(rows[u], acc_s.at[dis[q].at[kr]],
                                     ssem, add=True)

            @pl.when((kr == 2) & (p + 1 < NPHS))
            def _():
                for q in range(2):
                    @pl.when((p + 1) % 2 == q)
                    def _():
                        pltpu.async_copy(
                            src01_hbm.at[c, pl.ds(e0 + (p + 1) * PE, PE)],
                            sis[q], isem)
                        pltpu.async_copy(
                            dst_hbm.at[pl.ds(c0 + (p + 1) * CPP, CPP)],
                            dis[q], isem)

            @pl.when((kr == CPP - 2) & (p + 1 < NPHS))
            def _():
                pltpu.make_async_copy(src01_hbm.at[0, pl.ds(0, PE)], si0,
                                      isem).wait()
                pltpu.make_async_copy(dst_hbm.at[pl.ds(0, CPP)], di0,
                                      isem).wait()

            @pl.when(k + 2 < MAXC)
            def _():
                k2 = k + 2
                p2 = k2 // CPP
                o2 = (k2 - p2 * CPP) * ECS
                for q in range(2):
                    @pl.when(p2 % 2 == q)
                    def _():
                        pltpu.async_copy(
                            hflat_hbm.at[sis[q].at[pl.ds(o2, ECS)]],
                            rows[(u + 2) % 4], gsem)
        return carry

    lax.fori_loop(0, MAXC // 4, step, 0)
    pltpu.make_async_copy(hflat_hbm.at[pl.ds(0, ECS)], r0, ssem).wait()
    pltpu.make_async_copy(hflat_hbm.at[pl.ds(0, ECS)], r0, ssem).wait()
    plsc.subcore_barrier()
    pltpu.sync_copy(acc_s.at[pl.ds(s * RPT, RPT)],
                    out_hbm.at[c, pl.ds(s * RPT, RPT)])

    @pl.when(s == NS - 1)
    def _():
        pltpu.sync_copy(acc_s.at[pl.ds(0, XPAD)],
                        out_hbm.at[c, pl.ds(NPS, XPAD)])


_sc_spmm = functools.partial(
    pl.kernel,
    out_type=jax.ShapeDtypeStruct((2, NP, HALF), jnp.float32),
    compiler_params=pltpu.CompilerParams(use_tc_tiling_on_sc=False),
    mesh=plsc.VectorSubcoreMesh(core_axis_name="c", subcore_axis_name="s"),
    scratch_types=[
        pltpu.VMEM((CPP * ECS,), jnp.int32),
        pltpu.VMEM((CPP * ECS,), jnp.int32),
        pltpu.VMEM((CPP, ECS), jnp.int32),
        pltpu.VMEM((CPP, ECS), jnp.int32),
        pltpu.VMEM((ECS, HALF), jnp.float32),
        pltpu.VMEM((ECS, HALF), jnp.float32),
        pltpu.VMEM((ECS, HALF), jnp.float32),
        pltpu.VMEM((ECS, HALF), jnp.float32),
        pltpu.VMEM_SHARED((NPS, HALF), jnp.float32),
        pltpu.SemaphoreType.DMA,
        pltpu.SemaphoreType.DMA,
        pltpu.SemaphoreType.DMA,
    ],
)(_sc_spmm_body)


def _embed_body(atom_ref, chir_ref, embA_ref, embC_ref, h_ref):
    a = atom_ref[0, 0, :]
    ch = chir_ref[0, 0, :]
    oa = (a[:, None] == lax.broadcasted_iota(jnp.int32, (BLK, 128), 1)
          ).astype(jnp.float32)
    oc = (ch[:, None] == lax.broadcasted_iota(jnp.int32, (BLK, 8), 1)
          ).astype(jnp.float32)
    h = oa @ embA_ref[...] + oc @ embC_ref[...]
    h_ref[0] = h[:, :HALF]
    h_ref[1] = h[:, HALF:]


def _mlp_body(agg_ref, c9_ref, w1_ref, b1_ref, w2_ref, b2_ref, sc_ref,
              bb_ref, e9_ref, sl_ref, h_ref):
    cnt = c9_ref[0] + c9_ref[1]
    x = (jnp.concatenate([agg_ref[0], agg_ref[1]], axis=1)
         + cnt @ e9_ref[...] + sl_ref[...])
    hmid = jnp.maximum(x @ w1_ref[...] + b1_ref[...], 0.0)
    h = (hmid @ w2_ref[...] + b2_ref[...]) * sc_ref[...] + bb_ref[...]
    h = jnp.maximum(h, 0.0)
    h_ref[0] = h[:, :HALF]
    h_ref[1] = h[:, HALF:]


def _final_body(agg_ref, c9_ref, w1_ref, b1_ref, w2_ref, b2_ref, sc_ref,
                bb_ref, e9_ref, sl_ref, gid_ref, out_ref, sums_ref):
    i = pl.program_id(0)
    cnt = c9_ref[0] + c9_ref[1]
    x = (jnp.concatenate([agg_ref[0], agg_ref[1]], axis=1)
         + cnt @ e9_ref[...] + sl_ref[...])
    hmid = jnp.maximum(x @ w1_ref[...] + b1_ref[...], 0.0)
    h = (hmid @ w2_ref[...] + b2_ref[...]) * sc_ref[...] + bb_ref[...]
    lane = lax.broadcasted_iota(jnp.int32, (BLK, PD), 1)
    h_aug = jnp.where(lane == ONES_COL, 1.0, h)
    gids = gid_ref[0, 0, :]
    oh = (gids[:, None] == lax.broadcasted_iota(jnp.int32, (BLK, G), 1)
          ).astype(jnp.float32)
    part = lax.dot_general(oh, h_aug, (((0,), (0,)), ((), ())),
                           preferred_element_type=jnp.float32)

    @pl.when(i == 0)
    def _():
        sums_ref[...] = part

    @pl.when(i > 0)
    def _():
        sums_ref[...] = sums_ref[...] + part

    @pl.when(i == NBLK - 1)
    def _():
        sums = sums_ref[...]
        cntg = jnp.maximum(sums[:, ONES_COL:ONES_COL + 1], 1.0)
        avg = sums[:, 0:D] / cntg
        out_ref[...] = jnp.concatenate([avg, sums[:, 0:D]], axis=1)


def _full(shape):
    return pl.BlockSpec(shape, lambda i: tuple(0 for _ in shape))


_row3 = pl.BlockSpec((1, 1, BLK), lambda i: (i, 0, 0))
_hspec = pl.BlockSpec((2, BLK, HALF), lambda i: (0, i, 0))
_c9spec = pl.BlockSpec((2, BLK, 16), lambda i: (0, i, 0))

_tc_embed = pl.pallas_call(
    _embed_body,
    grid=(NBLK,),
    in_specs=[_row3, _row3, _full((128, PD)), _full((8, PD))],
    out_specs=_hspec,
    out_shape=jax.ShapeDtypeStruct((2, NP, HALF), jnp.float32),
)

_tc_mlp = pl.pallas_call(
    _mlp_body,
    grid=(NBLK,),
    in_specs=[_hspec, _c9spec, _full((PD, DH)), _full((1, DH)),
              _full((DH, PD)), _full((1, PD)), _full((1, PD)),
              _full((1, PD)), _full((16, PD)), _full((1, PD))],
    out_specs=_hspec,
    out_shape=jax.ShapeDtypeStruct((2, NP, HALF), jnp.float32),
)

_tc_final = pl.pallas_call(
    _final_body,
    grid=(NBLK,),
    in_specs=[_hspec, _c9spec, _full((PD, DH)), _full((1, DH)),
              _full((DH, PD)), _full((1, PD)), _full((1, PD)),
              _full((1, PD)), _full((16, PD)), _full((1, PD)), _row3],
    out_specs=pl.BlockSpec((G, 2 * D), lambda i: (0, 0)),
    out_shape=jax.ShapeDtypeStruct((G, 2 * D), jnp.float32),
    scratch_shapes=[pltpu.VMEM((G, PD), jnp.float32)],
)

_TAB = np.zeros((24, 16), np.float32)
for _b in range(6):
    for _d in range(3):
        _TAB[_b * 3 + _d, _b] = 1.0
        _TAB[_b * 3 + _d, 6 + _d] = 1.0


def kernel(atomic_number, chirality_type, edge_index, bond_type,
           bond_direction_type, graph_ids, node_emb_atomic,
           node_emb_chirality, edge_emb_bond, edge_emb_dir,
           W1, b1, W2, b2, bn_scale, bn_bias):
    f32, i32 = jnp.float32, jnp.int32
    pad = NP - N
    srcp = jnp.pad(edge_index[0].astype(i32), (0, EPS - E))
    src01 = jnp.stack([srcp, srcp + NP])
    dst = jnp.pad(edge_index[1].astype(i32), (0, EPS - E),
                  constant_values=SENT).reshape(NCHS, ECS)
    dstc = jnp.pad(edge_index[1].astype(i32), (0, EPC - E),
                   constant_values=SENT).reshape(NCHC, ECC)
    bond = jnp.pad(bond_type.astype(i32), (0, EPC - E))
    bdir = jnp.pad(bond_direction_type.astype(i32), (0, EPC - E))
    atom3 = jnp.pad(atomic_number.astype(i32), (0, pad)).reshape(NBLK, 1, BLK)
    chir3 = jnp.pad(chirality_type.astype(i32), (0, pad)).reshape(NBLK, 1, BLK)
    gid3 = jnp.pad(graph_ids.astype(i32), (0, pad),
                   constant_values=G + 7).reshape(NBLK, 1, BLK)
    embA = jnp.pad(node_emb_atomic.astype(f32), ((0, 8), (0, PD - D)))
    embC = jnp.pad(node_emb_chirality.astype(f32), ((0, 5), (0, PD - D)))
    W1p = jnp.pad(W1.astype(f32), ((0, 0), (0, PD - D), (0, DH - 2 * D)))
    b1p = jnp.pad(b1.astype(f32), ((0, 0), (0, DH - 2 * D))).reshape(NL, 1, DH)
    W2p = jnp.pad(W2.astype(f32), ((0, 0), (0, DH - 2 * D), (0, PD - D)))
    b2p = jnp.pad(b2.astype(f32), ((0, 0), (0, PD - D))).reshape(NL, 1, PD)
    scp = jnp.pad(bn_scale.astype(f32), ((0, 0), (0, PD - D))).reshape(NL, 1, PD)
    bbp = jnp.pad(bn_bias.astype(f32), ((0, 0), (0, PD - D))).reshape(NL, 1, PD)
    e9 = jnp.pad(
        jnp.concatenate([edge_emb_bond.astype(f32), edge_emb_dir.astype(f32)],
                        axis=1), ((0, 0), (0, 7), (0, PD - D)))
    slf = jnp.pad((edge_emb_bond[:, 4] + edge_emb_dir[:, 0]).astype(f32),
                  ((0, 0), (0, PD - D))).reshape(NL, 1, PD)
    tab = jnp.tile(jnp.asarray(_TAB), (32, 1))
    zeros16 = jnp.zeros((NP, 16), f32)

    counts = _sc_counts(bond, bdir, dstc, tab, zeros16)
    h2 = _tc_embed(atom3, chir3, embA, embC)
    for l in range(NL):
        agg = _sc_spmm(src01, dst, h2.reshape(2 * NP, HALF), h2)
        if l < NL - 1:
            h2 = _tc_mlp(agg, counts, W1p[l], b1p[l], W2p[l], b2p[l],
                         scp[l], bbp[l], e9[l], slf[l])
        else:
            res = _tc_final(agg, counts, W1p[l], b1p[l], W2p[l], b2p[l],
                            scp[l], bbp[l], e9[l], slf[l], gid3)
    return res
